# Initial kernel scaffold; baseline (speedup 1.0000x reference)
import jax
import jax.numpy as jnp
from jax.experimental import pallas as pl


def kernel(node_features, edge_index):
    raise NotImplementedError("write your pallas kernel here")


# speedup vs baseline: 1.9758x; 1.9758x over previous
import dataclasses
import functools

import jax
import jax.numpy as jnp
from jax import lax
from jax.experimental import pallas as pl
from jax.experimental.pallas import tpu as pltpu
from jax.experimental.pallas import tpu_sc as plsc

EPS = 1e-8
LANES = 16
NUM_CORES = 2
NUM_SUBCORES = 16
NW = NUM_CORES * NUM_SUBCORES
CHUNK = 128


def _normalize_body(x_ref, o_ref):
    x = x_ref[...]
    ss = jnp.sum(x * x, axis=1, keepdims=True)
    inv = 1.0 / jnp.maximum(jnp.sqrt(ss), EPS)
    o_ref[...] = x * inv


def _finalize_body(s_ref, d_ref, o_ref):
    s = jnp.sum(s_ref[...], axis=0, keepdims=True)
    d = jnp.sum(d_ref[...], axis=0, keepdims=True)
    o_ref[...] = jnp.where(d > 0.0, s / jnp.maximum(d, 1.0), 1.0)


@functools.lru_cache(maxsize=None)
def _make_edge_kernel(n_nodes, d, ch):
    nseg = d // LANES
    mesh = plsc.VectorSubcoreMesh(core_axis_name="c", subcore_axis_name="s")
    out_t = (
        jax.ShapeDtypeStruct((NUM_CORES, n_nodes), jnp.float32),
        jax.ShapeDtypeStruct((NUM_CORES, n_nodes), jnp.float32),
    )

    cp = pltpu.CompilerParams()
    if "needs_layout_passes" in pltpu.CompilerParams.__dataclass_fields__:
        cp = dataclasses.replace(cp, needs_layout_passes=False)

    @functools.partial(
        pl.kernel,
        out_type=out_t,
        mesh=mesh,
        compiler_params=cp,
        scratch_types=[
            pltpu.VMEM((ch, CHUNK), jnp.int32),
            pltpu.VMEM((ch, CHUNK), jnp.int32),
            pltpu.VMEM((ch, CHUNK), jnp.float32),
            pltpu.VMEM((CHUNK, d), jnp.float32),
            pltpu.VMEM((CHUNK, d), jnp.float32),
            pltpu.VMEM((CHUNK,), jnp.float32),
            pltpu.VMEM((n_nodes,), jnp.float32),
            pltpu.VMEM_SHARED((n_nodes,), jnp.float32),
            pltpu.VMEM_SHARED((n_nodes,), jnp.float32),
            pltpu.SemaphoreType.DMA,
            pltpu.SemaphoreType.DMA,
        ],
    )
    def edge_kernel(xhat_hbm, src_hbm, dst_hbm, valid_hbm, sums_hbm, degs_hbm,
                    src_v, dst_v, valid_v, srows, drows, sim_v, stage_v,
                    shared_sum, shared_deg, sem_a, sem_b):
        cid = lax.axis_index("c")
        sid = lax.axis_index("s")
        wid = sid * NUM_CORES + cid
        zeros16 = jnp.zeros((LANES,), jnp.float32)
        lane_iota = lax.iota(jnp.int32, LANES)

        @pl.when(sid == 0)
        def _init():
            @pl.loop(0, n_nodes, step=LANES)
            def _z(i):
                stage_v[pl.ds(pl.multiple_of(i, LANES), LANES)] = zeros16

            pltpu.sync_copy(stage_v, shared_sum)
            pltpu.sync_copy(stage_v, shared_deg)

        pltpu.sync_copy(src_hbm.at[wid], src_v)
        pltpu.sync_copy(dst_hbm.at[wid], dst_v)
        pltpu.sync_copy(valid_hbm.at[wid], valid_v)
        plsc.subcore_barrier()

        @pl.loop(0, ch)
        def _chunk(j):
            cp_a = pltpu.async_copy(xhat_hbm.at[src_v.at[j]], srows, sem_a)
            cp_b = pltpu.async_copy(xhat_hbm.at[dst_v.at[j]], drows, sem_b)
            cp_a.wait()
            cp_b.wait()

            @pl.loop(0, CHUNK // LANES)
            def _group(g):
                base = pl.multiple_of(g * LANES, LANES)
                sim_vec = zeros16
                for rr in range(LANES):
                    a = srows[base + rr, pl.ds(0, LANES)]
                    b = drows[base + rr, pl.ds(0, LANES)]
                    acc = a * b
                    for kk in range(1, nseg):
                        a = srows[base + rr, pl.ds(kk * LANES, LANES)]
                        b = drows[base + rr, pl.ds(kk * LANES, LANES)]
                        acc = acc + a * b
                    tot = jnp.sum(acc)
                    sim_vec = jnp.where(lane_iota == rr, tot, sim_vec)
                sim_v[pl.ds(base, LANES)] = sim_vec * valid_v[j, pl.ds(base, LANES)]

            pltpu.sync_copy(sim_v, shared_sum.at[src_v.at[j]], add=True)
            pltpu.sync_copy(sim_v, shared_sum.at[dst_v.at[j]], add=True)
            pltpu.sync_copy(valid_v.at[j], shared_deg.at[src_v.at[j]], add=True)
            pltpu.sync_copy(valid_v.at[j], shared_deg.at[dst_v.at[j]], add=True)

        plsc.subcore_barrier()

        @pl.when(sid == 0)
        def _drain():
            pltpu.sync_copy(shared_sum, stage_v)
            pltpu.sync_copy(stage_v, sums_hbm.at[cid])
            pltpu.sync_copy(shared_deg, stage_v)
            pltpu.sync_copy(stage_v, degs_hbm.at[cid])

    return edge_kernel


def kernel(node_features, edge_index):
    n, d = node_features.shape
    e = edge_index.shape[1]

    xhat = pl.pallas_call(
        _normalize_body,
        out_shape=jax.ShapeDtypeStruct((n, d), jnp.float32),
    )(node_features)

    ch = -(-e // (NW * CHUNK))
    ep = NW * CHUNK * ch
    pad = ep - e
    src = edge_index[0].astype(jnp.int32)
    dst = edge_index[1].astype(jnp.int32)
    srcp = jnp.pad(src, (0, pad)).reshape(NW, ch, CHUNK)
    dstp = jnp.pad(dst, (0, pad)).reshape(NW, ch, CHUNK)
    valid = jnp.pad(jnp.ones((e,), jnp.float32), (0, pad)).reshape(NW, ch, CHUNK)

    sums, degs = _make_edge_kernel(n, d, ch)(xhat, srcp, dstp, valid)

    out = pl.pallas_call(
        _finalize_body,
        out_shape=jax.ShapeDtypeStruct((1, n), jnp.float32),
    )(sums, degs)
    return out.reshape(n)


# speedup vs baseline: 3.3267x; 1.6837x over previous
import dataclasses
import functools

import jax
import jax.numpy as jnp
from jax import lax
from jax.experimental import pallas as pl
from jax.experimental.pallas import tpu as pltpu
from jax.experimental.pallas import tpu_sc as plsc

EPS = 1e-8
LANES = 16
NUM_CORES = 2
NUM_SUBCORES = 16
NW = NUM_CORES * NUM_SUBCORES
CHUNK = 64


def _normalize_body(x_ref, o_ref):
    x = x_ref[...]
    ss = jnp.sum(x * x, axis=1, keepdims=True)
    inv = 1.0 / jnp.maximum(jnp.sqrt(ss), EPS)
    o_ref[...] = x * inv


def _finalize_body(s_ref, d_ref, o_ref):
    s = jnp.sum(s_ref[...], axis=0, keepdims=True)
    d = jnp.sum(d_ref[...], axis=0, keepdims=True)
    o_ref[...] = jnp.where(d > 0.0, s / jnp.maximum(d, 1.0), 1.0)


@functools.lru_cache(maxsize=None)
def _make_edge_kernel(n_nodes, d, ch):
    nseg = d // LANES
    mesh = plsc.VectorSubcoreMesh(core_axis_name="c", subcore_axis_name="s")
    out_t = (
        jax.ShapeDtypeStruct((NUM_CORES, n_nodes), jnp.float32),
        jax.ShapeDtypeStruct((NUM_CORES, n_nodes), jnp.float32),
    )

    cp = pltpu.CompilerParams()
    if "needs_layout_passes" in pltpu.CompilerParams.__dataclass_fields__:
        cp = dataclasses.replace(cp, needs_layout_passes=False)

    @functools.partial(
        pl.kernel,
        out_type=out_t,
        mesh=mesh,
        compiler_params=cp,
        scratch_types=[
            pltpu.VMEM((ch, CHUNK), jnp.int32),
            pltpu.VMEM((ch, CHUNK), jnp.int32),
            pltpu.VMEM((ch, CHUNK), jnp.float32),
            pltpu.VMEM((2, CHUNK, d), jnp.float32),
            pltpu.VMEM((2, CHUNK, d), jnp.float32),
            pltpu.VMEM((CHUNK,), jnp.float32),
            pltpu.VMEM((n_nodes,), jnp.float32),
            pltpu.VMEM_SHARED((n_nodes,), jnp.float32),
            pltpu.VMEM_SHARED((n_nodes,), jnp.float32),
            pltpu.SemaphoreType.DMA,
            pltpu.SemaphoreType.DMA,
        ],
    )
    def edge_kernel(xhat_hbm, src_hbm, dst_hbm, valid_hbm, sums_hbm, degs_hbm,
                    src_v, dst_v, valid_v, srows, drows, sim_v, stage_v,
                    shared_sum, shared_deg, sem_a, sem_b):
        cid = lax.axis_index("c")
        sid = lax.axis_index("s")
        wid = sid * NUM_CORES + cid
        zeros16 = jnp.zeros((LANES,), jnp.float32)
        lane_iota = lax.iota(jnp.int32, LANES)

        @pl.when(sid == 0)
        def _init():
            @pl.loop(0, n_nodes, step=LANES)
            def _z(i):
                stage_v[pl.ds(pl.multiple_of(i, LANES), LANES)] = zeros16

            pltpu.sync_copy(stage_v, shared_sum)
            pltpu.sync_copy(stage_v, shared_deg)

        pltpu.sync_copy(src_hbm.at[wid], src_v)
        pltpu.sync_copy(dst_hbm.at[wid], dst_v)
        pltpu.sync_copy(valid_hbm.at[wid], valid_v)

        pltpu.async_copy(xhat_hbm.at[src_v.at[0]], srows.at[0], sem_a)
        pltpu.async_copy(xhat_hbm.at[dst_v.at[0]], drows.at[0], sem_b)
        plsc.subcore_barrier()

        @pl.loop(0, ch)
        def _chunk(j):
            par = lax.rem(j, 2)
            srow = srows.at[par]
            drow = drows.at[par]
            pltpu.make_async_copy(xhat_hbm.at[src_v.at[j]], srow, sem_a).wait()
            pltpu.make_async_copy(xhat_hbm.at[dst_v.at[j]], drow, sem_b).wait()

            @pl.when(j + 1 < ch)
            def _prefetch():
                nxt = 1 - par
                pltpu.async_copy(xhat_hbm.at[src_v.at[j + 1]], srows.at[nxt], sem_a)
                pltpu.async_copy(xhat_hbm.at[dst_v.at[j + 1]], drows.at[nxt], sem_b)

            @pl.loop(0, CHUNK // LANES)
            def _group(g):
                base = pl.multiple_of(g * LANES, LANES)
                sim_vec = zeros16
                for rr in range(LANES):
                    a = srow[base + rr, pl.ds(0, LANES)]
                    b = drow[base + rr, pl.ds(0, LANES)]
                    acc = a * b
                    for kk in range(1, nseg):
                        a = srow[base + rr, pl.ds(kk * LANES, LANES)]
                        b = drow[base + rr, pl.ds(kk * LANES, LANES)]
                        acc = acc + a * b
                    tot = jnp.sum(acc)
                    sim_vec = jnp.where(lane_iota == rr, tot, sim_vec)
                sim_v[pl.ds(base, LANES)] = sim_vec * valid_v[j, pl.ds(base, LANES)]

            pltpu.sync_copy(sim_v, shared_sum.at[src_v.at[j]], add=True)
            pltpu.sync_copy(sim_v, shared_sum.at[dst_v.at[j]], add=True)
            pltpu.sync_copy(valid_v.at[j], shared_deg.at[src_v.at[j]], add=True)
            pltpu.sync_copy(valid_v.at[j], shared_deg.at[dst_v.at[j]], add=True)

        plsc.subcore_barrier()

        @pl.when(sid == 0)
        def _drain():
            pltpu.sync_copy(shared_sum, stage_v)
            pltpu.sync_copy(stage_v, sums_hbm.at[cid])
            pltpu.sync_copy(shared_deg, stage_v)
            pltpu.sync_copy(stage_v, degs_hbm.at[cid])

    return edge_kernel


def kernel(node_features, edge_index):
    n, d = node_features.shape
    e = edge_index.shape[1]

    xhat = pl.pallas_call(
        _normalize_body,
        out_shape=jax.ShapeDtypeStruct((n, d), jnp.float32),
    )(node_features)

    ch = -(-e // (NW * CHUNK))
    ep = NW * CHUNK * ch
    pad = ep - e
    src = edge_index[0].astype(jnp.int32)
    dst = edge_index[1].astype(jnp.int32)
    srcp = jnp.pad(src, (0, pad)).reshape(NW, ch, CHUNK)
    dstp = jnp.pad(dst, (0, pad)).reshape(NW, ch, CHUNK)
    valid = jnp.pad(jnp.ones((e,), jnp.float32), (0, pad)).reshape(NW, ch, CHUNK)

    sums, degs = _make_edge_kernel(n, d, ch)(xhat, srcp, dstp, valid)

    out = pl.pallas_call(
        _finalize_body,
        out_shape=jax.ShapeDtypeStruct((1, n), jnp.float32),
    )(sums, degs)
    return out.reshape(n)
